# baseline (speedup 1.0000x reference)
import dataclasses
import functools

import jax
import jax.numpy as jnp
from jax import lax
from jax.experimental import pallas as pl
from jax.experimental.pallas import tpu as pltpu
from jax.experimental.pallas import tpu_sc as plsc

N = 10000
F = 128
E = 320000
NC = 2
NS = 16
NW = NC * NS
C = 80
NCHP = 128
EPAD = NW * NCHP * C
NPAD = 10240
STRIPE = NPAD // NS


def _sc_compiler_params():
    cp = pltpu.CompilerParams()
    if "needs_layout_passes" in pltpu.CompilerParams.__dataclass_fields__:
        cp = dataclasses.replace(cp, needs_layout_passes=False)
    return cp


_SC_MESH = dict(core_axis_name="c", subcore_axis_name="s")


def _dense_in_body(x_ref, w_ref, a_ref, h_ref, esd_ref):
    h = jnp.dot(x_ref[...], w_ref[...], preferred_element_type=jnp.float32,
                precision=lax.Precision.HIGHEST)
    h_ref[...] = h
    es = jnp.sum(h * a_ref[0][None, :], axis=1, keepdims=True)
    ed = jnp.sum(h * a_ref[1][None, :], axis=1, keepdims=True)
    esd_ref[...] = jnp.concatenate(
        [es, ed, jnp.zeros((N, F - 2), jnp.float32)], axis=1)


def _dense_in(x, W, a2):
    return pl.pallas_call(
        _dense_in_body,
        out_shape=[jax.ShapeDtypeStruct((N, F), jnp.float32),
                   jax.ShapeDtypeStruct((N, F), jnp.float32)],
    )(x, W, a2)


def _dense_mid_body(acc_ref, den_ref, w_ref, a_ref, h_ref, esd_ref):
    d = den_ref[0, :N] + den_ref[1, :N]
    y = (acc_ref[0, :N] + acc_ref[1, :N]) / (d + 1e-16)
    y = jnp.maximum(y, 0.0)
    h = jnp.dot(y, w_ref[...], preferred_element_type=jnp.float32,
                precision=lax.Precision.HIGHEST)
    h_ref[...] = h
    es = jnp.sum(h * a_ref[0][None, :], axis=1, keepdims=True)
    ed = jnp.sum(h * a_ref[1][None, :], axis=1, keepdims=True)
    esd_ref[...] = jnp.concatenate(
        [es, ed, jnp.zeros((N, F - 2), jnp.float32)], axis=1)


def _dense_mid(acc, den, W, a2):
    return pl.pallas_call(
        _dense_mid_body,
        out_shape=[jax.ShapeDtypeStruct((N, F), jnp.float32),
                   jax.ShapeDtypeStruct((N, F), jnp.float32)],
    )(acc, den, W, a2)


def _dense_out_body(acc_ref, den_ref, o_ref):
    d = den_ref[0, :N] + den_ref[1, :N]
    o_ref[...] = (acc_ref[0, :N] + acc_ref[1, :N]) / (d + 1e-16)


def _dense_out(acc, den):
    return pl.pallas_call(
        _dense_out_body,
        out_shape=jax.ShapeDtypeStruct((N, F), jnp.float32),
    )(acc, den)


def _edge_weights_sc(esd, src3, dst3):
    mesh = plsc.VectorSubcoreMesh(**_SC_MESH)

    @functools.partial(
        pl.kernel,
        out_type=[jax.ShapeDtypeStruct((NW, NCHP, C), jnp.float32),
                  jax.ShapeDtypeStruct((NC, NPAD), jnp.float32)],
        mesh=mesh,
        scratch_types=[
            pltpu.VMEM((NPAD,), jnp.float32),
            pltpu.VMEM((NPAD,), jnp.float32),
            pltpu.VMEM((NCHP, C), jnp.int32),
            pltpu.VMEM((NCHP, C), jnp.int32),
            pltpu.VMEM((NCHP, C), jnp.float32),
            pltpu.VMEM((STRIPE,), jnp.float32),
            pltpu.VMEM_SHARED((NPAD,), jnp.float32),
            pltpu.SemaphoreType.DMA,
        ],
        compiler_params=_sc_compiler_params(),
    )
    def k(esd_hbm, src_hbm, dst_hbm, ex_hbm, den_hbm,
          es_v, ed_v, src_v, dst_v, ex_v, zvec_v, den_sh, dsem):
        cid = lax.axis_index("c")
        sid = lax.axis_index("s")
        wid = sid * NC + cid

        pltpu.sync_copy(esd_hbm.at[0], es_v)
        pltpu.sync_copy(esd_hbm.at[1], ed_v)
        pltpu.sync_copy(src_hbm.at[wid], src_v)
        pltpu.sync_copy(dst_hbm.at[wid], dst_v)

        @pl.loop(0, STRIPE, step=16)
        def _(i):
            zvec_v[pl.ds(i, 16)] = jnp.zeros((16,), jnp.float32)

        base = sid * STRIPE
        pltpu.sync_copy(zvec_v, den_sh.at[pl.ds(base, STRIPE)])
        plsc.subcore_barrier()

        @pl.loop(0, NCHP)
        def _(j):
            @pl.loop(0, C, step=16)
            def _(s):
                sl = pl.ds(s, 16)
                s16 = src_v[j, sl]
                d16 = dst_v[j, sl]
                eg = (plsc.load_gather(es_v, [s16])
                      + plsc.load_gather(ed_v, [d16]))
                eg = jnp.maximum(eg, eg * 0.2)
                ex_v[j, sl] = jnp.exp(eg)

            pltpu.async_copy(ex_v.at[j], den_sh.at[dst_v.at[j]], dsem,
                             add=True)

        @pl.loop(0, NCHP)
        def _(j):
            pltpu.make_async_copy(ex_v.at[0], den_sh.at[pl.ds(0, C)],
                                  dsem).wait()

        pltpu.sync_copy(ex_v, ex_hbm.at[wid])
        plsc.subcore_barrier()
        pltpu.sync_copy(den_sh.at[pl.ds(base, STRIPE)],
                        den_hbm.at[cid, pl.ds(base, STRIPE)])

    return k(esd, src3, dst3)


def _edge_rows_sc(h, src3, dst3, ex3):
    mesh = plsc.VectorSubcoreMesh(**_SC_MESH)

    @functools.partial(
        pl.kernel,
        out_type=jax.ShapeDtypeStruct((NC, NPAD, F), jnp.float32),
        mesh=mesh,
        scratch_types=[
            pltpu.VMEM((4, C), jnp.int32),
            pltpu.VMEM((4, C), jnp.int32),
            pltpu.VMEM((4, C), jnp.float32),
            pltpu.VMEM((2, C, F), jnp.float32),
            pltpu.VMEM_SHARED((NPAD, F), jnp.float32),
            pltpu.SemaphoreType.DMA((4,)),
            pltpu.SemaphoreType.DMA((2,)),
            pltpu.SemaphoreType.DMA((2,)),
        ],
        compiler_params=_sc_compiler_params(),
    )
    def k(h_hbm, src_hbm, dst_hbm, ex_hbm, acc_hbm,
          src_v, dst_v, ex_v, rows_v, acc_sh, isem, gsem, ssem):
        cid = lax.axis_index("c")
        sid = lax.axis_index("s")
        wid = sid * NC + cid

        def start_idx(c, slot):
            pltpu.make_async_copy(src_hbm.at[wid, c], src_v.at[slot],
                                  isem.at[slot]).start()
            pltpu.make_async_copy(dst_hbm.at[wid, c], dst_v.at[slot],
                                  isem.at[slot]).start()
            pltpu.make_async_copy(ex_hbm.at[wid, c], ex_v.at[slot],
                                  isem.at[slot]).start()

        def wait_idx(slot):
            pltpu.make_async_copy(src_hbm.at[wid, 0], src_v.at[slot],
                                  isem.at[slot]).wait()
            pltpu.make_async_copy(dst_hbm.at[wid, 0], dst_v.at[slot],
                                  isem.at[slot]).wait()
            pltpu.make_async_copy(ex_hbm.at[wid, 0], ex_v.at[slot],
                                  isem.at[slot]).wait()

        def start_gather(islot, rslot):
            pltpu.make_async_copy(h_hbm.at[src_v.at[islot]],
                                  rows_v.at[rslot], gsem.at[rslot]).start()

        def wait_gather(islot, rslot):
            pltpu.make_async_copy(h_hbm.at[src_v.at[islot]],
                                  rows_v.at[rslot], gsem.at[rslot]).wait()

        def start_scatter(islot, rslot):
            pltpu.async_copy(rows_v.at[rslot], acc_sh.at[dst_v.at[islot]],
                             ssem.at[rslot], add=True)

        def wait_scatter(islot, rslot):
            pltpu.make_async_copy(rows_v.at[rslot],
                                  acc_sh.at[dst_v.at[islot]],
                                  ssem.at[rslot]).wait()

        @pl.loop(0, C)
        def _(r):
            @pl.loop(0, F, step=16)
            def _(v):
                rows_v[0, r, pl.ds(v, 16)] = jnp.zeros((16,), jnp.float32)

        base = sid * STRIPE

        @pl.loop(0, STRIPE, step=C)
        def _(r):
            pltpu.sync_copy(rows_v.at[0], acc_sh.at[pl.ds(base + r, C)])

        plsc.subcore_barrier()

        start_idx(0, 0)
        start_idx(1, 1)
        start_idx(2, 2)
        wait_idx(0)
        start_gather(0, 0)

        @pl.loop(0, NCHP, step=4)
        def _(j):
            for b in range(4):
                c = j + b
                b2 = b % 2
                o2 = 1 - b2
                oi = (b + 1) % 4

                @pl.when(c >= 1)
                def _():
                    wait_scatter((b + 3) % 4, o2)

                wait_gather(b, b2)

                @pl.when(c + 1 < NCHP)
                def _():
                    wait_idx(oi)
                    start_gather(oi, o2)

                @pl.when(c + 3 < NCHP)
                def _():
                    start_idx(c + 3, (b + 3) % 4)

                @plsc.parallel_loop(0, C, step=16)
                def _(s):
                    ex16 = ex_v[b, pl.ds(s, 16)]
                    for l in range(16):
                        a = ex16[l]
                        for v in range(F // 16):
                            slv = pl.ds(v * 16, 16)
                            rows_v[b2, s + l, slv] = (
                                rows_v[b2, s + l, slv] * a)

                start_scatter(b, b2)

        wait_scatter(3, 1)
        plsc.subcore_barrier()
        pltpu.sync_copy(acc_sh.at[pl.ds(base, STRIPE)],
                        acc_hbm.at[cid, pl.ds(base, STRIPE)])

    return k(h, src3, dst3, ex3)


def kernel(x, W1, a1_src, a1_dst, W2, a2_src, a2_dst, edge_index):
    ei = edge_index.astype(jnp.int32)
    ppw = NCHP * C - E // NW
    pad_src = jnp.zeros((NW, ppw), jnp.int32)
    pad_dst = jnp.broadcast_to(
        (N + jnp.arange(ppw, dtype=jnp.int32) % (NPAD - N))[None, :],
        (NW, ppw))
    srcp = jnp.concatenate(
        [ei[0].reshape(NW, E // NW), pad_src], axis=1).reshape(NW, NCHP, C)
    dstp = jnp.concatenate(
        [ei[1].reshape(NW, E // NW), pad_dst], axis=1).reshape(NW, NCHP, C)
    a1 = jnp.stack([a1_src, a1_dst])
    a2 = jnp.stack([a2_src, a2_dst])

    def layer(h, esd):
        esd = jnp.concatenate(
            [esd, jnp.zeros((2, NPAD - N), jnp.float32)], axis=1)
        ex3, den = _edge_weights_sc(esd, srcp, dstp)
        acc = _edge_rows_sc(h, srcp, dstp, ex3)
        return acc, den

    h1, esd1p = _dense_in(x, W1, a1)
    acc1, den1 = layer(h1, esd1p[:, :2].T)

    h2, esd2p = _dense_mid(acc1, den1[:, :, None], W2, a2)
    acc2, den2 = layer(h2, esd2p[:, :2].T)

    return _dense_out(acc2, den2[:, :, None])


# speedup vs baseline: 1.0623x; 1.0623x over previous
import dataclasses
import functools

import jax
import jax.numpy as jnp
from jax import lax
from jax.experimental import pallas as pl
from jax.experimental.pallas import tpu as pltpu
from jax.experimental.pallas import tpu_sc as plsc

N = 10000
F = 128
E = 320000
NC = 2
NS = 16
NW = NC * NS
C = 80
NCHP = 128
EPAD = NW * NCHP * C
NPAD = 10240
STRIPE = NPAD // NS


def _sc_compiler_params():
    cp = pltpu.CompilerParams()
    if "needs_layout_passes" in pltpu.CompilerParams.__dataclass_fields__:
        cp = dataclasses.replace(cp, needs_layout_passes=False)
    return cp


_SC_MESH = dict(core_axis_name="c", subcore_axis_name="s")


def _dense_in_body(x_ref, w_ref, a_ref, h_ref, esd_ref):
    h = jnp.dot(x_ref[...], w_ref[...], preferred_element_type=jnp.float32,
                precision=lax.Precision.HIGHEST)
    h_ref[...] = h
    es = jnp.sum(h * a_ref[0][None, :], axis=1, keepdims=True)
    ed = jnp.sum(h * a_ref[1][None, :], axis=1, keepdims=True)
    esd_ref[...] = jnp.concatenate(
        [es, ed, jnp.zeros((N, F - 2), jnp.float32)], axis=1)


def _dense_in(x, W, a2):
    return pl.pallas_call(
        _dense_in_body,
        out_shape=[jax.ShapeDtypeStruct((N, F), jnp.float32),
                   jax.ShapeDtypeStruct((N, F), jnp.float32)],
    )(x, W, a2)


def _dense_mid_body(acc_ref, den_ref, w_ref, a_ref, h_ref, esd_ref):
    d = den_ref[0, :N] + den_ref[1, :N]
    y = (acc_ref[0, :N] + acc_ref[1, :N]) / (d + 1e-16)
    y = jnp.maximum(y, 0.0)
    h = jnp.dot(y, w_ref[...], preferred_element_type=jnp.float32,
                precision=lax.Precision.HIGHEST)
    h_ref[...] = h
    es = jnp.sum(h * a_ref[0][None, :], axis=1, keepdims=True)
    ed = jnp.sum(h * a_ref[1][None, :], axis=1, keepdims=True)
    esd_ref[...] = jnp.concatenate(
        [es, ed, jnp.zeros((N, F - 2), jnp.float32)], axis=1)


def _dense_mid(acc, den, W, a2):
    return pl.pallas_call(
        _dense_mid_body,
        out_shape=[jax.ShapeDtypeStruct((N, F), jnp.float32),
                   jax.ShapeDtypeStruct((N, F), jnp.float32)],
    )(acc, den, W, a2)


def _dense_out_body(acc_ref, den_ref, o_ref):
    d = den_ref[0, :N] + den_ref[1, :N]
    o_ref[...] = (acc_ref[0, :N] + acc_ref[1, :N]) / (d + 1e-16)


def _dense_out(acc, den):
    return pl.pallas_call(
        _dense_out_body,
        out_shape=jax.ShapeDtypeStruct((N, F), jnp.float32),
    )(acc, den)


def _edge_weights_sc(esd, src3, dst3):
    mesh = plsc.VectorSubcoreMesh(**_SC_MESH)

    @functools.partial(
        pl.kernel,
        out_type=[jax.ShapeDtypeStruct((NW, NCHP, C), jnp.float32),
                  jax.ShapeDtypeStruct((NC, NPAD), jnp.float32)],
        mesh=mesh,
        scratch_types=[
            pltpu.VMEM((NPAD,), jnp.float32),
            pltpu.VMEM((NPAD,), jnp.float32),
            pltpu.VMEM((NCHP, C), jnp.int32),
            pltpu.VMEM((NCHP, C), jnp.int32),
            pltpu.VMEM((NCHP, C), jnp.float32),
            pltpu.VMEM((STRIPE,), jnp.float32),
            pltpu.VMEM_SHARED((NPAD,), jnp.float32),
            pltpu.SemaphoreType.DMA,
        ],
        compiler_params=_sc_compiler_params(),
    )
    def k(esd_hbm, src_hbm, dst_hbm, ex_hbm, den_hbm,
          es_v, ed_v, src_v, dst_v, ex_v, zvec_v, den_sh, dsem):
        cid = lax.axis_index("c")
        sid = lax.axis_index("s")
        wid = sid * NC + cid

        pltpu.sync_copy(esd_hbm.at[0], es_v)
        pltpu.sync_copy(esd_hbm.at[1], ed_v)
        pltpu.sync_copy(src_hbm.at[wid], src_v)
        pltpu.sync_copy(dst_hbm.at[wid], dst_v)

        @pl.loop(0, STRIPE, step=16)
        def _(i):
            zvec_v[pl.ds(i, 16)] = jnp.zeros((16,), jnp.float32)

        base = sid * STRIPE
        pltpu.sync_copy(zvec_v, den_sh.at[pl.ds(base, STRIPE)])
        plsc.subcore_barrier()

        @pl.loop(0, NCHP)
        def _(j):
            @pl.loop(0, C, step=16)
            def _(s):
                sl = pl.ds(s, 16)
                s16 = src_v[j, sl]
                d16 = dst_v[j, sl]
                eg = (plsc.load_gather(es_v, [s16])
                      + plsc.load_gather(ed_v, [d16]))
                eg = jnp.maximum(eg, eg * 0.2)
                ex_v[j, sl] = jnp.exp(eg)

            pltpu.async_copy(ex_v.at[j], den_sh.at[dst_v.at[j]], dsem,
                             add=True)

        @pl.loop(0, NCHP)
        def _(j):
            pltpu.make_async_copy(ex_v.at[0], den_sh.at[pl.ds(0, C)],
                                  dsem).wait()

        pltpu.sync_copy(ex_v, ex_hbm.at[wid])
        plsc.subcore_barrier()
        pltpu.sync_copy(den_sh.at[pl.ds(base, STRIPE)],
                        den_hbm.at[cid, pl.ds(base, STRIPE)])

    return k(esd, src3, dst3)


def _edge_rows_sc(h, src3, dst3, ex3):
    mesh = plsc.VectorSubcoreMesh(**_SC_MESH)

    @functools.partial(
        pl.kernel,
        out_type=jax.ShapeDtypeStruct((NC, NPAD, F), jnp.float32),
        mesh=mesh,
        scratch_types=[
            pltpu.VMEM((8, C), jnp.int32),
            pltpu.VMEM((8, C), jnp.int32),
            pltpu.VMEM((8, C), jnp.float32),
            pltpu.VMEM((4, C, F), jnp.float32),
            pltpu.VMEM_SHARED((NPAD, F), jnp.float32),
            pltpu.SemaphoreType.DMA((8,)),
            pltpu.SemaphoreType.DMA((4,)),
            pltpu.SemaphoreType.DMA((4,)),
        ],
        compiler_params=_sc_compiler_params(),
    )
    def k(h_hbm, src_hbm, dst_hbm, ex_hbm, acc_hbm,
          src_v, dst_v, ex_v, rows_v, acc_sh, isem, gsem, ssem):
        cid = lax.axis_index("c")
        sid = lax.axis_index("s")
        wid = sid * NC + cid

        def start_idx(c, slot):
            pltpu.make_async_copy(src_hbm.at[wid, c], src_v.at[slot],
                                  isem.at[slot]).start()
            pltpu.make_async_copy(dst_hbm.at[wid, c], dst_v.at[slot],
                                  isem.at[slot]).start()
            pltpu.make_async_copy(ex_hbm.at[wid, c], ex_v.at[slot],
                                  isem.at[slot]).start()

        def wait_idx(slot):
            pltpu.make_async_copy(src_hbm.at[wid, 0], src_v.at[slot],
                                  isem.at[slot]).wait()
            pltpu.make_async_copy(dst_hbm.at[wid, 0], dst_v.at[slot],
                                  isem.at[slot]).wait()
            pltpu.make_async_copy(ex_hbm.at[wid, 0], ex_v.at[slot],
                                  isem.at[slot]).wait()

        def start_gather(islot, rslot):
            pltpu.make_async_copy(h_hbm.at[src_v.at[islot]],
                                  rows_v.at[rslot], gsem.at[rslot]).start()

        def wait_gather(islot, rslot):
            pltpu.make_async_copy(h_hbm.at[src_v.at[islot]],
                                  rows_v.at[rslot], gsem.at[rslot]).wait()

        def start_scatter(islot, rslot):
            pltpu.async_copy(rows_v.at[rslot], acc_sh.at[dst_v.at[islot]],
                             ssem.at[rslot], add=True)

        def wait_scatter(islot, rslot):
            pltpu.make_async_copy(rows_v.at[rslot],
                                  acc_sh.at[dst_v.at[islot]],
                                  ssem.at[rslot]).wait()

        @pl.loop(0, C)
        def _(r):
            @pl.loop(0, F, step=16)
            def _(v):
                rows_v[0, r, pl.ds(v, 16)] = jnp.zeros((16,), jnp.float32)

        base = sid * STRIPE

        @pl.loop(0, STRIPE, step=C)
        def _(r):
            pltpu.sync_copy(rows_v.at[0], acc_sh.at[pl.ds(base + r, C)])

        plsc.subcore_barrier()

        for i in range(6):
            start_idx(i, i)
        wait_idx(0)
        start_gather(0, 0)
        wait_idx(1)
        start_gather(1, 1)

        @pl.loop(0, NCHP, step=8)
        def _(j):
            for b in range(8):
                c = j + b
                rb = b % 4

                @pl.when(c >= 2)
                def _():
                    wait_scatter((b + 6) % 8, (rb + 2) % 4)

                @pl.when(c + 2 < NCHP)
                def _():
                    wait_idx((b + 2) % 8)
                    start_gather((b + 2) % 8, (rb + 2) % 4)

                wait_gather(b, rb)

                @pl.when(c + 6 < NCHP)
                def _():
                    start_idx(c + 6, (b + 6) % 8)

                @plsc.parallel_loop(0, C, step=16)
                def _(s):
                    ex16 = ex_v[b, pl.ds(s, 16)]
                    for l in range(16):
                        a = ex16[l]
                        for v in range(F // 16):
                            slv = pl.ds(v * 16, 16)
                            rows_v[rb, s + l, slv] = (
                                rows_v[rb, s + l, slv] * a)

                start_scatter(b, rb)

        wait_scatter(6, 2)
        wait_scatter(7, 3)
        plsc.subcore_barrier()
        pltpu.sync_copy(acc_sh.at[pl.ds(base, STRIPE)],
                        acc_hbm.at[cid, pl.ds(base, STRIPE)])

    return k(h, src3, dst3, ex3)


def kernel(x, W1, a1_src, a1_dst, W2, a2_src, a2_dst, edge_index):
    ei = edge_index.astype(jnp.int32)
    ppw = NCHP * C - E // NW
    pad_src = jnp.zeros((NW, ppw), jnp.int32)
    pad_dst = jnp.broadcast_to(
        (N + jnp.arange(ppw, dtype=jnp.int32) % (NPAD - N))[None, :],
        (NW, ppw))
    srcp = jnp.concatenate(
        [ei[0].reshape(NW, E // NW), pad_src], axis=1).reshape(NW, NCHP, C)
    dstp = jnp.concatenate(
        [ei[1].reshape(NW, E // NW), pad_dst], axis=1).reshape(NW, NCHP, C)
    a1 = jnp.stack([a1_src, a1_dst])
    a2 = jnp.stack([a2_src, a2_dst])

    def layer(h, esd):
        esd = jnp.concatenate(
            [esd, jnp.zeros((2, NPAD - N), jnp.float32)], axis=1)
        ex3, den = _edge_weights_sc(esd, srcp, dstp)
        acc = _edge_rows_sc(h, srcp, dstp, ex3)
        return acc, den

    h1, esd1p = _dense_in(x, W1, a1)
    acc1, den1 = layer(h1, esd1p[:, :2].T)

    h2, esd2p = _dense_mid(acc1, den1[:, :, None], W2, a2)
    acc2, den2 = layer(h2, esd2p[:, :2].T)

    return _dense_out(acc2, den2[:, :, None])
